# Initial kernel scaffold; baseline (speedup 1.0000x reference)
import jax
import jax.numpy as jnp
from jax.experimental import pallas as pl


def kernel(x0, x1, ch_mask, w0, b0, w1, b1, w2, b2, w3, b3, wc, bc):
    raise NotImplementedError("write your pallas kernel here")


# speedup vs baseline: 2.0989x; 2.0989x over previous
import jax
import jax.numpy as jnp
from jax.experimental import pallas as pl
from jax.experimental.pallas import tpu as pltpu

NS = 4
C = 96
H = 224
W = 224
BH = 32
NBLK = H // BH

_f32 = jnp.float32


def _gumbel_cm(ch_mask):
    u = jax.random.uniform(jax.random.key(1234), ch_mask.shape,
                           minval=1e-6, maxval=1.0 - 1e-6, dtype=_f32)
    g = -jnp.log(-jnp.log(u))
    return jax.nn.softmax((ch_mask + g) / 1.0, axis=3)


def _rows8(*vs):
    pad = [jnp.zeros((C,), _f32)] * (8 - len(vs))
    return jnp.stack(list(vs) + pad)


def _dot(a, b):
    return jax.lax.dot_general(a, b, (((1,), (0,)), ((), ())),
                               preferred_element_type=_f32)


def _stage0_kernel(xp_ref, spa_ref, w_ref, cv_ref, out_ref):
    r0 = pl.program_id(0) * BH
    acc = jnp.zeros((BH * W, C), _f32)
    for dh in range(3):
        for dw in range(3):
            xs = xp_ref[pl.ds(r0 + dh, BH), pl.ds(dw, W), :].reshape(BH * W, C)
            acc += _dot(xs, w_ref[dh * 3 + dw])
    spa = spa_ref[...]
    u = cv_ref[0, :]
    v = cv_ref[1, :]
    b = cv_ref[2, :]
    t = acc.reshape(BH, W, C) + b
    fea = t * (u * spa + v)
    out_ref[...] = jnp.maximum(fea, 0.0)


def _mid_stage_kernel(xp_ref, spa_ref, wc_ref, wd_ref, cv_ref, out_ref):
    r0 = pl.program_id(0) * BH
    acc_c = jnp.zeros((BH * W, C), _f32)
    acc_d = jnp.zeros((BH * W, C), _f32)
    for dh in range(3):
        for dw in range(3):
            xs = xp_ref[pl.ds(r0 + dh, BH), pl.ds(dw, W), :].reshape(BH * W, C)
            k = dh * 3 + dw
            acc_c += _dot(xs, wc_ref[k])
            acc_d += _dot(xs, wd_ref[k])
    spa = spa_ref[...]
    a1 = cv_ref[0, :]
    tc = cv_ref[1, :]
    ts = cv_ref[2, :]
    t = acc_d.reshape(BH, W, C) * a1
    fea = spa * (acc_c.reshape(BH, W, C) + ts - t) + t + tc
    out_ref[...] = jnp.maximum(fea, 0.0)


def _final_kernel(f0_ref, f1_ref, f2_ref, f3_ref, w_ref, bc_ref, out_ref):
    acc = jnp.zeros((BH * W, C), _f32)
    for i, f in enumerate((f0_ref, f1_ref, f2_ref, f3_ref)):
        acc += _dot(f[...].reshape(BH * W, C), w_ref[i])
    out_ref[...] = (acc + bc_ref[0, :]).reshape(BH, W, C)


_GRID = (NBLK,)
_XSPEC = pl.BlockSpec((H + 2, W + 2, C), lambda i: (0, 0, 0))
_SPASPEC = pl.BlockSpec((BH, W, 1), lambda i: (i, 0, 0))
_W9SPEC = pl.BlockSpec((9, C, C), lambda i: (0, 0, 0))
_W4SPEC = pl.BlockSpec((NS, C, C), lambda i: (0, 0, 0))
_CVSPEC = pl.BlockSpec((8, C), lambda i: (0, 0))
_OSPEC = pl.BlockSpec((BH, W, C), lambda i: (i, 0, 0))
_OSHAPE = jax.ShapeDtypeStruct((H, W, C), _f32)
_CP = pltpu.CompilerParams(vmem_limit_bytes=100 * 1024 * 1024)


def kernel(x0, x1, ch_mask, w0, b0, w1, b1, w2, b2, w3, b3, wc, bc):
    cm = _gumbel_cm(ch_mask)
    spa = jnp.transpose(x1[0], (1, 2, 0))
    x = jnp.transpose(x0[0], (1, 2, 0))
    xp = jnp.pad(x, ((1, 1), (1, 1), (0, 0)))

    w0k = jnp.transpose(w0, (2, 3, 1, 0)).reshape(9, C, C)
    cv0 = _rows8(cm[0, :, 0, 0], cm[0, :, 0, 1], b0)
    fea = pl.pallas_call(
        _stage0_kernel, grid=_GRID,
        in_specs=[_XSPEC, _SPASPEC, _W9SPEC, _CVSPEC],
        out_specs=_OSPEC, out_shape=_OSHAPE, compiler_params=_CP,
    )(xp, spa, w0k, cv0)
    outs = [fea]

    ws = (w1, w2, w3)
    bs = (b1, b2, b3)
    for i in range(1, NS):
        wik = jnp.transpose(ws[i - 1], (2, 3, 1, 0))
        d = cm[0, :, i - 1, 1]
        wck = wik.reshape(9, C, C)
        wdk = (wik * d[None, None, :, None]).reshape(9, C, C)
        a0 = cm[0, :, i, 0]
        a1 = cm[0, :, i, 1]
        bi = bs[i - 1]
        cv = _rows8(a1, bi * a1, bi * (a0 + 1.0))
        xpi = jnp.pad(fea, ((1, 1), (1, 1), (0, 0)))
        fea = pl.pallas_call(
            _mid_stage_kernel, grid=_GRID,
            in_specs=[_XSPEC, _SPASPEC, _W9SPEC, _W9SPEC, _CVSPEC],
            out_specs=_OSPEC, out_shape=_OSHAPE, compiler_params=_CP,
        )(xpi, spa, wck, wdk, cv)
        outs.append(fea)

    w4 = jnp.transpose(wc.reshape(C, NS, C), (1, 2, 0))
    bcv = _rows8(bc)
    y = pl.pallas_call(
        _final_kernel, grid=_GRID,
        in_specs=[_OSPEC] * NS + [_W4SPEC, _CVSPEC],
        out_specs=_OSPEC, out_shape=_OSHAPE, compiler_params=_CP,
    )(*outs, w4, bcv)
    y = jnp.transpose(y, (2, 0, 1))[None]
    return y, cm


# speedup vs baseline: 2.3330x; 1.1115x over previous
import jax
import jax.numpy as jnp
from jax.experimental import pallas as pl
from jax.experimental.pallas import tpu as pltpu

NS = 4
C = 96
H = 224
W = 224
BH = 32
NBLK = H // BH

_f32 = jnp.float32
_bf16 = jnp.bfloat16


def _gumbel_cm(ch_mask):
    u = jax.random.uniform(jax.random.key(1234), ch_mask.shape,
                           minval=1e-6, maxval=1.0 - 1e-6, dtype=_f32)
    g = -jnp.log(-jnp.log(u))
    return jax.nn.softmax((ch_mask + g) / 1.0, axis=3)


def _rows8(*vs):
    pad = [jnp.zeros((C,), _f32)] * (8 - len(vs))
    return jnp.stack(list(vs) + pad)


def _dot(a, b):
    return jax.lax.dot_general(a, b, (((1,), (0,)), ((), ())),
                               preferred_element_type=_f32)


def _stage0_kernel(xp_ref, spa_ref, w_ref, cv_ref, out_ref):
    r0 = pl.program_id(0) * BH
    acc = jnp.zeros((BH * W, C), _f32)
    for dh in range(3):
        for dw in range(3):
            xs = xp_ref[pl.ds(r0 + dh, BH), pl.ds(dw, W), :].reshape(BH * W, C)
            acc += _dot(xs, w_ref[dh * 3 + dw])
    spa = spa_ref[...]
    u = cv_ref[0, :]
    v = cv_ref[1, :]
    b = cv_ref[2, :]
    t = acc.reshape(BH, W, C) + b
    fea = t * (u * spa + v)
    out_ref[...] = jnp.maximum(fea, 0.0).astype(_bf16)


def _mid_stage_kernel(xp_ref, spa_ref, wc_ref, wd_ref, cv_ref, out_ref):
    r0 = pl.program_id(0) * BH
    acc_c = jnp.zeros((BH * W, C), _f32)
    acc_d = jnp.zeros((BH * W, C), _f32)
    for dh in range(3):
        for dw in range(3):
            xs = xp_ref[pl.ds(r0 + dh, BH), pl.ds(dw, W), :].reshape(BH * W, C)
            k = dh * 3 + dw
            acc_c += _dot(xs, wc_ref[k])
            acc_d += _dot(xs, wd_ref[k])
    spa = spa_ref[...]
    a1 = cv_ref[0, :]
    tc = cv_ref[1, :]
    ts = cv_ref[2, :]
    t = acc_d.reshape(BH, W, C) * a1
    fea = spa * (acc_c.reshape(BH, W, C) + ts - t) + t + tc
    out_ref[...] = jnp.maximum(fea, 0.0).astype(_bf16)


def _final_kernel(f0_ref, f1_ref, f2_ref, f3_ref, w_ref, bc_ref, out_ref):
    acc = jnp.zeros((BH * W, C), _f32)
    for i, f in enumerate((f0_ref, f1_ref, f2_ref, f3_ref)):
        acc += _dot(f[...].reshape(BH * W, C), w_ref[i])
    out_ref[...] = (acc + bc_ref[0, :]).reshape(BH, W, C)


_GRID = (NBLK,)
_XSPEC = pl.BlockSpec((H + 2, W + 2, C), lambda i: (0, 0, 0))
_SPASPEC = pl.BlockSpec((BH, W, 1), lambda i: (i, 0, 0))
_W9SPEC = pl.BlockSpec((9, C, C), lambda i: (0, 0, 0))
_W4SPEC = pl.BlockSpec((NS, C, C), lambda i: (0, 0, 0))
_CVSPEC = pl.BlockSpec((8, C), lambda i: (0, 0))
_OSPEC = pl.BlockSpec((BH, W, C), lambda i: (i, 0, 0))
_OSHAPE = jax.ShapeDtypeStruct((H, W, C), _bf16)
_YSHAPE = jax.ShapeDtypeStruct((H, W, C), _f32)
_CP = pltpu.CompilerParams(vmem_limit_bytes=100 * 1024 * 1024)


def kernel(x0, x1, ch_mask, w0, b0, w1, b1, w2, b2, w3, b3, wc, bc):
    cm = _gumbel_cm(ch_mask)
    spa = jnp.transpose(x1[0], (1, 2, 0))
    x = jnp.transpose(x0[0], (1, 2, 0)).astype(_bf16)
    xp = jnp.pad(x, ((1, 1), (1, 1), (0, 0)))

    w0k = jnp.transpose(w0, (2, 3, 1, 0)).reshape(9, C, C).astype(_bf16)
    cv0 = _rows8(cm[0, :, 0, 0], cm[0, :, 0, 1], b0)
    fea = pl.pallas_call(
        _stage0_kernel, grid=_GRID,
        in_specs=[_XSPEC, _SPASPEC, _W9SPEC, _CVSPEC],
        out_specs=_OSPEC, out_shape=_OSHAPE, compiler_params=_CP,
    )(xp, spa, w0k, cv0)
    outs = [fea]

    ws = (w1, w2, w3)
    bs = (b1, b2, b3)
    for i in range(1, NS):
        wik = jnp.transpose(ws[i - 1], (2, 3, 1, 0))
        d = cm[0, :, i - 1, 1]
        wck = wik.reshape(9, C, C).astype(_bf16)
        wdk = (wik * d[None, None, :, None]).reshape(9, C, C).astype(_bf16)
        a0 = cm[0, :, i, 0]
        a1 = cm[0, :, i, 1]
        bi = bs[i - 1]
        cv = _rows8(a1, bi * a1, bi * (a0 + 1.0))
        xpi = jnp.pad(fea, ((1, 1), (1, 1), (0, 0)))
        fea = pl.pallas_call(
            _mid_stage_kernel, grid=_GRID,
            in_specs=[_XSPEC, _SPASPEC, _W9SPEC, _W9SPEC, _CVSPEC],
            out_specs=_OSPEC, out_shape=_OSHAPE, compiler_params=_CP,
        )(xpi, spa, wck, wdk, cv)
        outs.append(fea)

    w4 = jnp.transpose(wc.reshape(C, NS, C), (1, 2, 0)).astype(_bf16)
    bcv = _rows8(bc)
    y = pl.pallas_call(
        _final_kernel, grid=_GRID,
        in_specs=[_OSPEC] * NS + [_W4SPEC, _CVSPEC],
        out_specs=_OSPEC, out_shape=_YSHAPE, compiler_params=_CP,
    )(*outs, w4, bcv)
    y = jnp.transpose(y, (2, 0, 1))[None]
    return y, cm
